# baseline (speedup 1.0000x reference)
import functools

import jax
import jax.numpy as jnp
from jax import lax
from jax.experimental import pallas as pl
from jax.experimental.pallas import tpu as pltpu
from jax.experimental.pallas import tpu_sc as plsc

ROWS = 16384
COLS = 4096
LANES = 16

SC_ROWS = 13056
NUM_WORKERS = 32
ROWS_PER_WORKER = SC_ROWS // NUM_WORKERS
R = 2
CHUNK = R * COLS
NUM_CHUNKS = ROWS_PER_WORKER // R
NVEC = COLS // LANES
NBUF = 4

TC_ROWS = ROWS - SC_ROWS
LB = 128
NB = COLS // LB
BLOCK_ROWS = 8
TC_GRID = TC_ROWS // BLOCK_ROWS

_mesh = plsc.VectorSubcoreMesh(core_axis_name="c", subcore_axis_name="s")

_scratch = (
    [pltpu.VMEM((COLS,), jnp.int32)]
    + [pltpu.VMEM((CHUNK,), jnp.float32) for _ in range(2 * NBUF)]
    + [pltpu.SemaphoreType.DMA for _ in range(2 * NBUF)]
)


@functools.partial(
    pl.kernel,
    out_type=jax.ShapeDtypeStruct((ROWS * COLS,), jnp.float32),
    mesh=_mesh,
    compiler_params=pltpu.CompilerParams(needs_layout_passes=False),
    scratch_types=_scratch,
)
def _permute_sc(x_hbm, perm_hbm, out_hbm, perm_v, *bufs_and_sems):
    ins = bufs_and_sems[0:NBUF]
    obs = bufs_and_sems[NBUF:2 * NBUF]
    isems = bufs_and_sems[2 * NBUF:3 * NBUF]
    osems = bufs_and_sems[3 * NBUF:4 * NBUF]

    wid = lax.axis_index("s") * 2 + lax.axis_index("c")
    base = wid * ROWS_PER_WORKER * COLS

    pltpu.sync_copy(perm_hbm, perm_v)

    def in_off(g):
        return base + jnp.minimum(g, NUM_CHUNKS - 1) * CHUNK

    def gather(src, dst):
        @plsc.parallel_loop(0, NVEC, unroll=8)
        def _(j):
            idxv = perm_v[pl.ds(j * LANES, LANES)]
            for r in range(R):
                v = plsc.load_gather(src, [idxv + (r * COLS)])
                dst[pl.ds(r * COLS + j * LANES, LANES)] = v

    for b in range(NBUF):
        pltpu.async_copy(x_hbm.at[pl.ds(base + b * CHUNK, CHUNK)],
                         ins[b], isems[b])

    def ring_body(go, _):
        for b in range(NBUF):
            g = go + b
            pltpu.make_async_copy(x_hbm.at[pl.ds(base, CHUNK)],
                                  ins[b], isems[b]).wait()

            @pl.when(go > 0)
            def _():
                pltpu.make_async_copy(obs[b],
                                      out_hbm.at[pl.ds(base, CHUNK)],
                                      osems[b]).wait()

            gather(ins[b], obs[b])
            pltpu.async_copy(obs[b],
                             out_hbm.at[pl.ds(base + g * CHUNK, CHUNK)],
                             osems[b])
            pltpu.async_copy(x_hbm.at[pl.ds(in_off(g + NBUF), CHUNK)],
                             ins[b], isems[b])
        return 0

    lax.fori_loop(0, NUM_CHUNKS // NBUF,
                  lambda go, c: ring_body(go * NBUF, c), 0)

    for b in range(NBUF):
        pltpu.make_async_copy(x_hbm.at[pl.ds(base, CHUNK)],
                              ins[b], isems[b]).wait()
        pltpu.make_async_copy(obs[b], out_hbm.at[pl.ds(base, CHUNK)],
                              osems[b]).wait()


def _tc_body(owner_ref, lane_ref, x_ref, o_ref):
    xs = [x_ref[:, a * LB:(a + 1) * LB] for a in range(NB)]
    for j in range(NB):
        own_j = owner_ref[0:1, j * LB:(j + 1) * LB]
        lane_j = jnp.broadcast_to(lane_ref[0:1, j * LB:(j + 1) * LB],
                                  (BLOCK_ROWS, LB))
        acc = jnp.zeros((BLOCK_ROWS, LB), jnp.float32)
        for a in range(NB):
            g = jnp.take_along_axis(xs[a], lane_j, axis=1)
            acc = jnp.where(own_j == a, g, acc)
        o_ref[:, j * LB:(j + 1) * LB] = acc


def _permute_tc(owner, lane, x):
    return pl.pallas_call(
        _tc_body,
        out_shape=jax.ShapeDtypeStruct((TC_ROWS, COLS), jnp.float32),
        grid=(TC_GRID,),
        in_specs=[
            pl.BlockSpec((1, COLS), lambda i: (0, 0)),
            pl.BlockSpec((1, COLS), lambda i: (0, 0)),
            pl.BlockSpec((BLOCK_ROWS, COLS),
                         lambda i: (SC_ROWS // BLOCK_ROWS + i, 0)),
        ],
        out_specs=pl.BlockSpec((BLOCK_ROWS, COLS), lambda i: (i, 0)),
    )(owner, lane, x)


def kernel(x, perm, perm_inv):
    del perm_inv
    p = perm.astype(jnp.int32)
    owner = (p // LB).reshape(1, COLS)
    lane = (p % LB).reshape(1, COLS)
    sc_out = _permute_sc(x.reshape(-1), p).reshape(ROWS, COLS)
    tc_out = _permute_tc(owner, lane, x)
    return lax.dynamic_update_slice(sc_out, tc_out, (SC_ROWS, 0))


# speedup vs baseline: 1.6064x; 1.6064x over previous
import functools

import jax
import jax.numpy as jnp
from jax import lax
from jax.experimental import pallas as pl
from jax.experimental.pallas import tpu as pltpu
from jax.experimental.pallas import tpu_sc as plsc

ROWS = 16384
COLS = 4096
LANES = 16
NUM_WORKERS = 32
ROWS_PER_WORKER = ROWS // NUM_WORKERS
RIN = 8
ROUT = 2
IN_CHUNK = RIN * COLS
OUT_CHUNK = ROUT * COLS
NUM_IN_CHUNKS = ROWS_PER_WORKER // RIN
SUB = RIN // ROUT
NVEC = COLS // LANES
NIBUF = 2
NOBUF = SUB

_mesh = plsc.VectorSubcoreMesh(core_axis_name="c", subcore_axis_name="s")

_scratch = (
    [pltpu.VMEM((COLS,), jnp.int32)]
    + [pltpu.VMEM((IN_CHUNK,), jnp.float32) for _ in range(NIBUF)]
    + [pltpu.VMEM((OUT_CHUNK,), jnp.float32) for _ in range(NOBUF)]
    + [pltpu.SemaphoreType.DMA for _ in range(NIBUF + NOBUF)]
)


@functools.partial(
    pl.kernel,
    out_type=jax.ShapeDtypeStruct((ROWS * COLS,), jnp.float32),
    mesh=_mesh,
    compiler_params=pltpu.CompilerParams(needs_layout_passes=False),
    scratch_types=_scratch,
)
def _permute_sc(x_hbm, perm_hbm, out_hbm, perm_v, *bufs_and_sems):
    ins = bufs_and_sems[0:NIBUF]
    obs = bufs_and_sems[NIBUF:NIBUF + NOBUF]
    isems = bufs_and_sems[NIBUF + NOBUF:2 * NIBUF + NOBUF]
    osems = bufs_and_sems[2 * NIBUF + NOBUF:2 * NIBUF + 2 * NOBUF]

    wid = lax.axis_index("s") * 2 + lax.axis_index("c")
    base = wid * ROWS_PER_WORKER * COLS

    pltpu.sync_copy(perm_hbm, perm_v)

    def gather2(src, dst, row0):
        @plsc.parallel_loop(0, NVEC, unroll=8)
        def _(j):
            idxv = perm_v[pl.ds(j * LANES, LANES)]
            for r in range(ROUT):
                v = plsc.load_gather(src, [idxv + ((row0 + r) * COLS)])
                dst[pl.ds(r * COLS + j * LANES, LANES)] = v

    for b in range(NIBUF):
        pltpu.async_copy(x_hbm.at[pl.ds(base + b * IN_CHUNK, IN_CHUNK)],
                         ins[b], isems[b])

    def pair_body(go, _):
        for ib in range(NIBUF):
            g = go + ib
            pltpu.make_async_copy(x_hbm.at[pl.ds(base, IN_CHUNK)],
                                  ins[ib], isems[ib]).wait()
            for h in range(SUB):
                @pl.when(g > 0)
                def _():
                    pltpu.make_async_copy(obs[h],
                                          out_hbm.at[pl.ds(base, OUT_CHUNK)],
                                          osems[h]).wait()

                gather2(ins[ib], obs[h], h * ROUT)
                q_off = base + g * IN_CHUNK + h * OUT_CHUNK
                pltpu.async_copy(obs[h], out_hbm.at[pl.ds(q_off, OUT_CHUNK)],
                                 osems[h])
            nxt = base + jnp.minimum(g + NIBUF, NUM_IN_CHUNKS - 1) * IN_CHUNK
            pltpu.async_copy(x_hbm.at[pl.ds(nxt, IN_CHUNK)], ins[ib],
                             isems[ib])
        return 0

    lax.fori_loop(0, NUM_IN_CHUNKS // NIBUF,
                  lambda go, c: pair_body(go * NIBUF, c), 0)

    for b in range(NIBUF):
        pltpu.make_async_copy(x_hbm.at[pl.ds(base, IN_CHUNK)],
                              ins[b], isems[b]).wait()
    for h in range(NOBUF):
        pltpu.make_async_copy(obs[h], out_hbm.at[pl.ds(base, OUT_CHUNK)],
                              osems[h]).wait()


def kernel(x, perm, perm_inv):
    del perm_inv
    out_flat = _permute_sc(x.reshape(-1), perm.astype(jnp.int32))
    return out_flat.reshape(ROWS, COLS)


# speedup vs baseline: 1.6084x; 1.0013x over previous
import functools

import jax
import jax.numpy as jnp
from jax import lax
from jax.experimental import pallas as pl
from jax.experimental.pallas import tpu as pltpu
from jax.experimental.pallas import tpu_sc as plsc

ROWS = 16384
COLS = 4096
LANES = 16
NUM_WORKERS = 32
ROWS_PER_WORKER = ROWS // NUM_WORKERS
R = 2
CHUNK = R * COLS
NUM_CHUNKS = ROWS_PER_WORKER // R
NVEC = COLS // LANES
NBUF = 4

_mesh = plsc.VectorSubcoreMesh(core_axis_name="c", subcore_axis_name="s")

_scratch = (
    [pltpu.VMEM((COLS,), jnp.int32)]
    + [pltpu.VMEM((CHUNK,), jnp.float32) for _ in range(2 * NBUF)]
    + [pltpu.SemaphoreType.DMA for _ in range(2 * NBUF)]
)


@functools.partial(
    pl.kernel,
    out_type=jax.ShapeDtypeStruct((ROWS * COLS,), jnp.float32),
    mesh=_mesh,
    compiler_params=pltpu.CompilerParams(needs_layout_passes=False),
    scratch_types=_scratch,
)
def _permute_sc(x_hbm, perm_hbm, out_hbm, perm_v, *bufs_and_sems):
    ins = bufs_and_sems[0:NBUF]
    obs = bufs_and_sems[NBUF:2 * NBUF]
    isems = bufs_and_sems[2 * NBUF:3 * NBUF]
    osems = bufs_and_sems[3 * NBUF:4 * NBUF]

    wid = lax.axis_index("s") * 2 + lax.axis_index("c")
    base = wid * ROWS_PER_WORKER * COLS

    pltpu.sync_copy(perm_hbm, perm_v)

    def in_off(g):
        return base + jnp.minimum(g, NUM_CHUNKS - 1) * CHUNK

    def gather(src, dst):
        @plsc.parallel_loop(0, NVEC, unroll=8)
        def _(j):
            idxv = perm_v[pl.ds(j * LANES, LANES)]
            for r in range(R):
                v = plsc.load_gather(src, [idxv + (r * COLS)])
                dst[pl.ds(r * COLS + j * LANES, LANES)] = v

    for b in range(NBUF):
        pltpu.async_copy(x_hbm.at[pl.ds(base + b * CHUNK, CHUNK)],
                         ins[b], isems[b])

    def ring_body(go, _):
        for b in range(NBUF):
            g = go + b
            pltpu.make_async_copy(x_hbm.at[pl.ds(base, CHUNK)],
                                  ins[b], isems[b]).wait()

            @pl.when(go > 0)
            def _():
                pltpu.make_async_copy(obs[b],
                                      out_hbm.at[pl.ds(base, CHUNK)],
                                      osems[b]).wait()

            gather(ins[b], obs[b])
            pltpu.async_copy(obs[b],
                             out_hbm.at[pl.ds(base + g * CHUNK, CHUNK)],
                             osems[b])
            pltpu.async_copy(x_hbm.at[pl.ds(in_off(g + NBUF), CHUNK)],
                             ins[b], isems[b])
        return 0

    lax.fori_loop(0, NUM_CHUNKS // NBUF, lambda go, c: ring_body(go * NBUF, c), 0)

    for b in range(NBUF):
        pltpu.make_async_copy(x_hbm.at[pl.ds(base, CHUNK)],
                              ins[b], isems[b]).wait()
        pltpu.make_async_copy(obs[b], out_hbm.at[pl.ds(base, CHUNK)],
                              osems[b]).wait()


def kernel(x, perm, perm_inv):
    del perm_inv
    out_flat = _permute_sc(x.reshape(-1), perm.astype(jnp.int32))
    return out_flat.reshape(ROWS, COLS)
